# Initial kernel scaffold; baseline (speedup 1.0000x reference)
import jax
import jax.numpy as jnp
from jax.experimental import pallas as pl


def kernel(x, edge_index, batch, params):
    raise NotImplementedError("write your pallas kernel here")


# speedup vs baseline: 7.5709x; 7.5709x over previous
import functools

import jax
import jax.numpy as jnp
from jax import lax
from jax.experimental import pallas as pl
from jax.experimental.pallas import tpu as pltpu
from jax.experimental.pallas import tpu_sc as plsc

_N = 10000
_E = 320000
_F_IN = 128
_HID = 64
_NCLS = 2
_L = 3
_NG = 64

_SC_CORES = 2
_SC_TILES = 16
_NW = _SC_CORES * _SC_TILES
_CHUNK = 125
_EW = _E // _NW
_IT = _EW // _CHUNK
_NPAD = 10240
_RPT = _NPAD // _SC_TILES


def _make_agg(F):
  mesh = plsc.VectorSubcoreMesh(core_axis_name="c", subcore_axis_name="s")

  @functools.partial(
      pl.kernel,
      out_type=jax.ShapeDtypeStruct((_SC_CORES, _NPAD, F), jnp.float32),
      mesh=mesh,
      compiler_params=pltpu.CompilerParams(use_tc_tiling_on_sc=False),
      scratch_types=[
          pltpu.VMEM((_IT, _CHUNK), jnp.int32),
          pltpu.VMEM((_IT, _CHUNK), jnp.int32),
          pltpu.VMEM((_CHUNK, F), jnp.float32),
          pltpu.VMEM_SHARED((_NPAD, F), jnp.float32),
          pltpu.SemaphoreType.DMA,
      ],
  )
  def agg(h_hbm, src_hbm, dst_hbm, zeros_hbm, out_hbm,
          src_v, dst_v, rows_v, acc_sh, sem):
    c = lax.axis_index("c")
    s = lax.axis_index("s")
    w = c * _SC_TILES + s
    pltpu.sync_copy(zeros_hbm.at[pl.ds(s * _RPT, _RPT)],
                    acc_sh.at[pl.ds(s * _RPT, _RPT)])
    pltpu.sync_copy(src_hbm.at[pl.ds(w * _IT, _IT)], src_v)
    pltpu.sync_copy(dst_hbm.at[pl.ds(w * _IT, _IT)], dst_v)
    plsc.subcore_barrier()

    def body(i, carry):
      pltpu.async_copy(h_hbm.at[src_v.at[i]], rows_v, sem).wait()
      pltpu.sync_copy(rows_v, acc_sh.at[dst_v.at[i]], add=True)
      return carry

    lax.fori_loop(0, _IT, body, 0)
    plsc.subcore_barrier()
    pltpu.sync_copy(acc_sh.at[pl.ds(s * _RPT, _RPT)],
                    out_hbm.at[c, pl.ds(s * _RPT, _RPT)])

  return agg


_agg128 = _make_agg(_F_IN)
_agg64 = _make_agg(_HID)


def _mlp_body(h_ref, a_ref, w1_ref, b1_ref, w2_ref, b2_ref, g_ref, be_ref,
              o_ref):
  z = h_ref[...] + a_ref[0, :_N] + a_ref[1, :_N]
  z = jnp.dot(z, w1_ref[...], preferred_element_type=jnp.float32, precision=lax.Precision.HIGHEST) + b1_ref[...]
  z = jnp.maximum(z, 0.0)
  z = jnp.dot(z, w2_ref[...], preferred_element_type=jnp.float32, precision=lax.Precision.HIGHEST) + b2_ref[...]
  mean = jnp.mean(z, axis=0, keepdims=True)
  zc = z - mean
  var = jnp.mean(zc * zc, axis=0, keepdims=True)
  zn = zc * lax.rsqrt(var + 1e-5)
  o_ref[...] = jnp.maximum(zn * g_ref[...] + be_ref[...], 0.0)


def _mlp_call(h, agg, w1, b1, w2, b2, gamma, beta):
  return pl.pallas_call(
      _mlp_body,
      out_shape=jax.ShapeDtypeStruct((_N, _HID), jnp.float32),
  )(h, agg, w1, b1.reshape(1, -1), w2, b2.reshape(1, -1),
    gamma.reshape(1, -1), beta.reshape(1, -1))


def _final_body(h1_ref, h2_ref, h3_ref, b_ref, wj_ref, bj_ref, wc1_ref,
                bc1_ref, wc2_ref, bc2_ref, o_ref):
  gids = lax.broadcasted_iota(jnp.int32, (_NG, _N), 0)
  mask = (gids == b_ref[...]).astype(jnp.float32)
  p1 = jnp.dot(mask, h1_ref[...], preferred_element_type=jnp.float32, precision=lax.Precision.HIGHEST)
  p2 = jnp.dot(mask, h2_ref[...], preferred_element_type=jnp.float32, precision=lax.Precision.HIGHEST)
  p3 = jnp.dot(mask, h3_ref[...], preferred_element_type=jnp.float32, precision=lax.Precision.HIGHEST)
  counts = jnp.sum(mask, axis=1, keepdims=True)
  pooled = (jnp.dot(p1, wj_ref[0:_HID], preferred_element_type=jnp.float32, precision=lax.Precision.HIGHEST)
            + jnp.dot(p2, wj_ref[_HID:2 * _HID],
                      preferred_element_type=jnp.float32, precision=lax.Precision.HIGHEST)
            + jnp.dot(p3, wj_ref[2 * _HID:3 * _HID],
                      preferred_element_type=jnp.float32, precision=lax.Precision.HIGHEST)
            + counts * bj_ref[...])
  cmid = jnp.maximum(
      jnp.dot(pooled, wc1_ref[...], preferred_element_type=jnp.float32, precision=lax.Precision.HIGHEST)
      + bc1_ref[...], 0.0)
  o_ref[...] = (jnp.dot(cmid, wc2_ref[...], preferred_element_type=jnp.float32, precision=lax.Precision.HIGHEST)
                + bc2_ref[...])


def _final_call(h1, h2, h3, batch2d, wj, bj, wc1, bc1, wc2, bc2):
  return pl.pallas_call(
      _final_body,
      out_shape=jax.ShapeDtypeStruct((_NG, _NCLS), jnp.float32),
  )(h1, h2, h3, batch2d, wj, bj.reshape(1, -1), wc1, bc1.reshape(1, -1),
    wc2, bc2.reshape(1, -1))


def kernel(x, edge_index, batch, params):
  src2d = edge_index[0].reshape(_E // _CHUNK, _CHUNK)
  dst2d = edge_index[1].reshape(_E // _CHUNK, _CHUNK)
  batch2d = batch.reshape(1, _N)
  zeros128 = jnp.zeros((_NPAD, _F_IN), jnp.float32)
  zeros64 = jnp.zeros((_NPAD, _HID), jnp.float32)
  h = x
  reps = []
  for i in range(_L):
    aggfn = _agg128 if i == 0 else _agg64
    zeros = zeros128 if i == 0 else zeros64
    agg = aggfn(h, src2d, dst2d, zeros)
    h = _mlp_call(h, agg, params['W1_%d' % i], params['b1_%d' % i],
                  params['W2_%d' % i], params['b2_%d' % i],
                  params['gamma_%d' % i], params['beta_%d' % i])
    reps.append(h)
  return _final_call(reps[0], reps[1], reps[2], batch2d,
                     params['Wj'], params['bj'], params['Wc1'],
                     params['bc1'], params['Wc2'], params['bc2'])


# speedup vs baseline: 11.5165x; 1.5212x over previous
import functools

import jax
import jax.numpy as jnp
from jax import lax
from jax.experimental import pallas as pl
from jax.experimental.pallas import tpu as pltpu
from jax.experimental.pallas import tpu_sc as plsc

_N = 10000
_E = 320000
_F_IN = 128
_HID = 64
_NCLS = 2
_L = 3
_NG = 64

_SC_CORES = 2
_SC_TILES = 16
_NW = _SC_CORES * _SC_TILES
_CHUNK = 100
_EW = _E // _NW
_IT = _EW // _CHUNK
_NPAD = 10240
_RPT = _NPAD // _SC_TILES


def _make_agg(F):
  mesh = plsc.VectorSubcoreMesh(core_axis_name="c", subcore_axis_name="s")

  @functools.partial(
      pl.kernel,
      out_type=jax.ShapeDtypeStruct((_SC_CORES, _NPAD, F), jnp.float32),
      mesh=mesh,
      compiler_params=pltpu.CompilerParams(use_tc_tiling_on_sc=False),
      scratch_types=[
          pltpu.VMEM((_IT, _CHUNK), jnp.int32),
          pltpu.VMEM((_IT, _CHUNK), jnp.int32),
          pltpu.VMEM((_CHUNK, F), jnp.float32),
          pltpu.VMEM((_CHUNK, F), jnp.float32),
          pltpu.VMEM_SHARED((_NPAD, F), jnp.float32),
          pltpu.SemaphoreType.DMA,
          pltpu.SemaphoreType.DMA,
      ],
  )
  def agg(h_hbm, src_hbm, dst_hbm, zeros_hbm, out_hbm,
          src_v, dst_v, rows0_v, rows1_v, acc_sh, sem0, sem1):
    c = lax.axis_index("c")
    s = lax.axis_index("s")
    w = c * _SC_TILES + s
    pltpu.sync_copy(zeros_hbm.at[pl.ds(s * _RPT, _RPT)],
                    acc_sh.at[pl.ds(s * _RPT, _RPT)])
    pltpu.sync_copy(src_hbm.at[pl.ds(w * _IT, _IT)], src_v)
    pltpu.sync_copy(dst_hbm.at[pl.ds(w * _IT, _IT)], dst_v)
    plsc.subcore_barrier()

    pltpu.async_copy(h_hbm.at[src_v.at[0]], rows0_v, sem0)

    def body(j, carry):
      i0 = 2 * j
      i1 = i0 + 1
      pltpu.async_copy(h_hbm.at[src_v.at[i1]], rows1_v, sem1)
      pltpu.make_async_copy(h_hbm.at[src_v.at[0]], rows0_v, sem0).wait()
      pltpu.sync_copy(rows0_v, acc_sh.at[dst_v.at[i0]], add=True)
      nxt = lax.rem(i0 + 2, _IT)
      pltpu.async_copy(h_hbm.at[src_v.at[nxt]], rows0_v, sem0)
      pltpu.make_async_copy(h_hbm.at[src_v.at[0]], rows1_v, sem1).wait()
      pltpu.sync_copy(rows1_v, acc_sh.at[dst_v.at[i1]], add=True)
      return carry

    lax.fori_loop(0, _IT // 2, body, 0)
    pltpu.make_async_copy(h_hbm.at[src_v.at[0]], rows0_v, sem0).wait()
    plsc.subcore_barrier()
    pltpu.sync_copy(acc_sh.at[pl.ds(s * _RPT, _RPT)],
                    out_hbm.at[c, pl.ds(s * _RPT, _RPT)])

  return agg


_agg128 = _make_agg(_F_IN)
_agg64 = _make_agg(_HID)


def _mlp_body(h_ref, a_ref, w1_ref, b1_ref, w2_ref, b2_ref, g_ref, be_ref,
              o_ref):
  z = h_ref[...] + a_ref[0, :_N] + a_ref[1, :_N]
  z = jnp.dot(z, w1_ref[...], preferred_element_type=jnp.float32) + b1_ref[...]
  z = jnp.maximum(z, 0.0)
  z = jnp.dot(z, w2_ref[...], preferred_element_type=jnp.float32) + b2_ref[...]
  mean = jnp.mean(z, axis=0, keepdims=True)
  zc = z - mean
  var = jnp.mean(zc * zc, axis=0, keepdims=True)
  zn = zc / jnp.sqrt(var + 1e-5)
  o_ref[...] = jnp.maximum(zn * g_ref[...] + be_ref[...], 0.0)


def _mlp_call(h, agg, w1, b1, w2, b2, gamma, beta):
  return pl.pallas_call(
      _mlp_body,
      out_shape=jax.ShapeDtypeStruct((_N, _HID), jnp.float32),
  )(h, agg, w1, b1.reshape(1, -1), w2, b2.reshape(1, -1),
    gamma.reshape(1, -1), beta.reshape(1, -1))


def _final_body(h1_ref, h2_ref, h3_ref, b_ref, wj_ref, bj_ref, wc1_ref,
                bc1_ref, wc2_ref, bc2_ref, o_ref):
  hc = jnp.concatenate([h1_ref[...], h2_ref[...], h3_ref[...]], axis=1)
  hj = jnp.dot(hc, wj_ref[...], preferred_element_type=jnp.float32) + bj_ref[...]
  gids = lax.broadcasted_iota(jnp.int32, (_NG, _N), 0)
  mask = (gids == b_ref[...]).astype(jnp.float32)
  pooled = jnp.dot(mask, hj, preferred_element_type=jnp.float32,
                   precision=lax.Precision.HIGHEST)
  cmid = jnp.maximum(
      jnp.dot(pooled, wc1_ref[...], preferred_element_type=jnp.float32)
      + bc1_ref[...], 0.0)
  o_ref[...] = (jnp.dot(cmid, wc2_ref[...], preferred_element_type=jnp.float32)
                + bc2_ref[...])


def _final_call(h1, h2, h3, batch2d, wj, bj, wc1, bc1, wc2, bc2):
  return pl.pallas_call(
      _final_body,
      out_shape=jax.ShapeDtypeStruct((_NG, _NCLS), jnp.float32),
  )(h1, h2, h3, batch2d, wj, bj.reshape(1, -1), wc1, bc1.reshape(1, -1),
    wc2, bc2.reshape(1, -1))


def kernel(x, edge_index, batch, params):
  src2d = edge_index[0].reshape(_E // _CHUNK, _CHUNK)
  dst2d = edge_index[1].reshape(_E // _CHUNK, _CHUNK)
  batch2d = batch.reshape(1, _N)
  zeros128 = jnp.zeros((_NPAD, _F_IN), jnp.float32)
  zeros64 = jnp.zeros((_NPAD, _HID), jnp.float32)
  h = x
  reps = []
  for i in range(_L):
    aggfn = _agg128 if i == 0 else _agg64
    zeros = zeros128 if i == 0 else zeros64
    agg = aggfn(h, src2d, dst2d, zeros)
    h = _mlp_call(h, agg, params['W1_%d' % i], params['b1_%d' % i],
                  params['W2_%d' % i], params['b2_%d' % i],
                  params['gamma_%d' % i], params['beta_%d' % i])
    reps.append(h)
  return _final_call(reps[0], reps[1], reps[2], batch2d,
                     params['Wj'], params['bj'], params['Wc1'],
                     params['bc1'], params['Wc2'], params['bc2'])


# speedup vs baseline: 12.9120x; 1.1212x over previous
import functools

import jax
import jax.numpy as jnp
from jax import lax
from jax.experimental import pallas as pl
from jax.experimental.pallas import tpu as pltpu
from jax.experimental.pallas import tpu_sc as plsc

_N = 10000
_E = 320000
_F_IN = 128
_HID = 64
_NCLS = 2
_L = 3
_NG = 64

_SC_CORES = 2
_SC_TILES = 16
_NW = _SC_CORES * _SC_TILES
_EW = _E // _NW
_NPAD = 10240
_RPT = _NPAD // _SC_TILES
_CHUNK128 = 100
_CHUNK64 = 125


def _make_agg(F, chunk, nbuf):
  it = _EW // chunk
  mesh = plsc.VectorSubcoreMesh(core_axis_name="c", subcore_axis_name="s")

  @functools.partial(
      pl.kernel,
      out_type=jax.ShapeDtypeStruct((_SC_CORES, _NPAD, F), jnp.float32),
      mesh=mesh,
      compiler_params=pltpu.CompilerParams(use_tc_tiling_on_sc=False),
      scratch_types=(
          [pltpu.VMEM((it, chunk), jnp.int32)] * 2
          + [pltpu.VMEM((chunk, F), jnp.float32)] * nbuf
          + [pltpu.VMEM_SHARED((_NPAD, F), jnp.float32)]
          + [pltpu.SemaphoreType.DMA] * nbuf
      ),
  )
  def agg(h_hbm, src_hbm, dst_hbm, zeros_hbm, out_hbm, src_v, dst_v, *rest):
    rows = rest[:nbuf]
    acc_sh = rest[nbuf]
    sems = rest[nbuf + 1:]
    c = lax.axis_index("c")
    s = lax.axis_index("s")
    w = c * _SC_TILES + s
    pltpu.sync_copy(zeros_hbm.at[pl.ds(s * _RPT, _RPT)],
                    acc_sh.at[pl.ds(s * _RPT, _RPT)])
    pltpu.sync_copy(src_hbm.at[pl.ds(w * it, it)], src_v)
    pltpu.sync_copy(dst_hbm.at[pl.ds(w * it, it)], dst_v)
    plsc.subcore_barrier()

    for b in range(nbuf):
      pltpu.async_copy(h_hbm.at[src_v.at[b]], rows[b], sems[b])

    def body(j, carry):
      for k in range(nbuf):
        i = nbuf * j + k
        pltpu.make_async_copy(h_hbm.at[src_v.at[0]], rows[k], sems[k]).wait()
        pltpu.sync_copy(rows[k], acc_sh.at[dst_v.at[i]], add=True)
        nxt = lax.rem(i + nbuf, it)
        pltpu.async_copy(h_hbm.at[src_v.at[nxt]], rows[k], sems[k])
      return carry

    lax.fori_loop(0, it // nbuf, body, 0)
    for b in range(nbuf):
      pltpu.make_async_copy(h_hbm.at[src_v.at[0]], rows[b], sems[b]).wait()
    plsc.subcore_barrier()
    pltpu.sync_copy(acc_sh.at[pl.ds(s * _RPT, _RPT)],
                    out_hbm.at[c, pl.ds(s * _RPT, _RPT)])

  return agg


_agg128 = _make_agg(_F_IN, _CHUNK128, 2)
_agg64 = _make_agg(_HID, _CHUNK64, 4)


def _mlp(h, a0, a1, w1, b1, w2, b2, g, be):
  z = h + a0 + a1
  z = jnp.dot(z, w1, preferred_element_type=jnp.float32) + b1
  z = jnp.maximum(z, 0.0)
  z = jnp.dot(z, w2, preferred_element_type=jnp.float32) + b2
  mean = jnp.mean(z, axis=0, keepdims=True)
  zc = z - mean
  var = jnp.mean(zc * zc, axis=0, keepdims=True)
  zn = zc / jnp.sqrt(var + 1e-5)
  return jnp.maximum(zn * g + be, 0.0)


def _mlp_body(h_ref, a_ref, w1_ref, b1_ref, w2_ref, b2_ref, g_ref, be_ref,
              o_ref):
  o_ref[...] = _mlp(h_ref[...], a_ref[0, :_N], a_ref[1, :_N], w1_ref[...],
                    b1_ref[...], w2_ref[...], b2_ref[...], g_ref[...],
                    be_ref[...])


def _mlp_call(h, agg, w1, b1, w2, b2, gamma, beta):
  return pl.pallas_call(
      _mlp_body,
      out_shape=jax.ShapeDtypeStruct((_N, _HID), jnp.float32),
  )(h, agg, w1, b1.reshape(1, -1), w2, b2.reshape(1, -1),
    gamma.reshape(1, -1), beta.reshape(1, -1))


def _last_body(h_ref, a_ref, w1_ref, b1_ref, w2_ref, b2_ref, g_ref, be_ref,
               h1_ref, h2_ref, b2d_ref, wj_ref, bj_ref, wc1_ref, bc1_ref,
               wc2_ref, bc2_ref, o_ref):
  h3 = _mlp(h_ref[...], a_ref[0, :_N], a_ref[1, :_N], w1_ref[...],
            b1_ref[...], w2_ref[...], b2_ref[...], g_ref[...], be_ref[...])
  hc = jnp.concatenate([h1_ref[...], h2_ref[...], h3], axis=1)
  hj = jnp.dot(hc, wj_ref[...], preferred_element_type=jnp.float32) + bj_ref[...]
  gids = lax.broadcasted_iota(jnp.int32, (_NG, _N), 0)
  mask = (gids == b2d_ref[...]).astype(jnp.float32)
  pooled = jnp.dot(mask, hj, preferred_element_type=jnp.float32,
                   precision=lax.Precision.HIGHEST)
  cmid = jnp.maximum(
      jnp.dot(pooled, wc1_ref[...], preferred_element_type=jnp.float32)
      + bc1_ref[...], 0.0)
  o_ref[...] = (jnp.dot(cmid, wc2_ref[...], preferred_element_type=jnp.float32)
                + bc2_ref[...])


def _last_call(h, agg, w1, b1, w2, b2, gamma, beta, h1, h2, batch2d, wj, bj,
               wc1, bc1, wc2, bc2):
  return pl.pallas_call(
      _last_body,
      out_shape=jax.ShapeDtypeStruct((_NG, _NCLS), jnp.float32),
  )(h, agg, w1, b1.reshape(1, -1), w2, b2.reshape(1, -1),
    gamma.reshape(1, -1), beta.reshape(1, -1), h1, h2, batch2d, wj,
    bj.reshape(1, -1), wc1, bc1.reshape(1, -1), wc2, bc2.reshape(1, -1))


def kernel(x, edge_index, batch, params):
  src128 = edge_index[0].reshape(_E // _CHUNK128, _CHUNK128)
  dst128 = edge_index[1].reshape(_E // _CHUNK128, _CHUNK128)
  src64 = edge_index[0].reshape(_E // _CHUNK64, _CHUNK64)
  dst64 = edge_index[1].reshape(_E // _CHUNK64, _CHUNK64)
  batch2d = batch.reshape(1, _N)
  zeros128 = jnp.zeros((_NPAD, _F_IN), jnp.float32)
  zeros64 = jnp.zeros((_NPAD, _HID), jnp.float32)

  agg = _agg128(x, src128, dst128, zeros128)
  h1 = _mlp_call(x, agg, params['W1_0'], params['b1_0'], params['W2_0'],
                 params['b2_0'], params['gamma_0'], params['beta_0'])
  agg = _agg64(h1, src64, dst64, zeros64)
  h2 = _mlp_call(h1, agg, params['W1_1'], params['b1_1'], params['W2_1'],
                 params['b2_1'], params['gamma_1'], params['beta_1'])
  agg = _agg64(h2, src64, dst64, zeros64)
  return _last_call(h2, agg, params['W1_2'], params['b1_2'], params['W2_2'],
                    params['b2_2'], params['gamma_2'], params['beta_2'],
                    h1, h2, batch2d, params['Wj'], params['bj'],
                    params['Wc1'], params['bc1'], params['Wc2'],
                    params['bc2'])


# speedup vs baseline: 13.6267x; 1.0554x over previous
import functools

import jax
import jax.numpy as jnp
from jax import lax
from jax.experimental import pallas as pl
from jax.experimental.pallas import tpu as pltpu
from jax.experimental.pallas import tpu_sc as plsc

_N = 10000
_E = 320000
_F_IN = 128
_HID = 64
_NCLS = 2
_L = 3
_NG = 64

_SC_CORES = 2
_SC_TILES = 16
_NW = _SC_CORES * _SC_TILES
_EW = _E // _NW
_NPAD = 10240
_RPT = _NPAD // _SC_TILES
_CHUNK = 100
_IT = _EW // _CHUNK


def _make_agg(F, nbuf):
  chunk, it = _CHUNK, _IT
  mesh = plsc.VectorSubcoreMesh(core_axis_name="c", subcore_axis_name="s")

  @functools.partial(
      pl.kernel,
      out_type=jax.ShapeDtypeStruct((_SC_CORES, _NPAD, F), jnp.float32),
      mesh=mesh,
      compiler_params=pltpu.CompilerParams(use_tc_tiling_on_sc=False),
      scratch_types=(
          [pltpu.VMEM((it, chunk), jnp.int32)] * 2
          + [pltpu.VMEM((chunk, F), jnp.float32)] * nbuf
          + [pltpu.VMEM_SHARED((_NPAD, F), jnp.float32)]
          + [pltpu.SemaphoreType.DMA] * nbuf
      ),
  )
  def agg(h_hbm, edge_hbm, zeros_hbm, out_hbm, src_v, dst_v, *rest):
    rows = rest[:nbuf]
    acc_sh = rest[nbuf]
    sems = rest[nbuf + 1:]
    c = lax.axis_index("c")
    s = lax.axis_index("s")
    w = c * _SC_TILES + s
    pltpu.sync_copy(zeros_hbm.at[pl.ds(s * _RPT, _RPT)],
                    acc_sh.at[pl.ds(s * _RPT, _RPT)])
    pltpu.sync_copy(edge_hbm.at[0, pl.ds(w * it, it)], src_v)
    pltpu.sync_copy(edge_hbm.at[1, pl.ds(w * it, it)], dst_v)
    plsc.subcore_barrier()

    for b in range(nbuf):
      pltpu.async_copy(h_hbm.at[src_v.at[b]], rows[b], sems[b])

    def body(j, carry):
      for k in range(nbuf):
        i = nbuf * j + k
        pltpu.make_async_copy(h_hbm.at[src_v.at[0]], rows[k], sems[k]).wait()
        pltpu.sync_copy(rows[k], acc_sh.at[dst_v.at[i]], add=True)
        nxt = lax.rem(i + nbuf, it)
        pltpu.async_copy(h_hbm.at[src_v.at[nxt]], rows[k], sems[k])
      return carry

    lax.fori_loop(0, it // nbuf, body, 0)
    for b in range(nbuf):
      pltpu.make_async_copy(h_hbm.at[src_v.at[0]], rows[b], sems[b]).wait()
    plsc.subcore_barrier()
    pltpu.sync_copy(acc_sh.at[pl.ds(s * _RPT, _RPT)],
                    out_hbm.at[c, pl.ds(s * _RPT, _RPT)])

  return agg


_agg128 = _make_agg(_F_IN, 2)
_agg64 = _make_agg(_HID, 6)


def _mlp(h, a0, a1, w1, b1, w2, b2, g, be):
  z = h + a0 + a1
  z = jnp.dot(z, w1, preferred_element_type=jnp.float32) + b1
  z = jnp.maximum(z, 0.0)
  z = jnp.dot(z, w2, preferred_element_type=jnp.float32) + b2
  mean = jnp.mean(z, axis=0, keepdims=True)
  zc = z - mean
  var = jnp.mean(zc * zc, axis=0, keepdims=True)
  zn = zc / jnp.sqrt(var + 1e-5)
  return jnp.maximum(zn * g + be, 0.0)


def _mlp_body(h_ref, a_ref, w1_ref, b1_ref, w2_ref, b2_ref, g_ref, be_ref,
              o_ref):
  o_ref[...] = _mlp(h_ref[...], a_ref[0, :_N], a_ref[1, :_N], w1_ref[...],
                    b1_ref[...], w2_ref[...], b2_ref[...], g_ref[...],
                    be_ref[...])


def _mlp_call(h, agg, w1, b1, w2, b2, gamma, beta):
  return pl.pallas_call(
      _mlp_body,
      out_shape=jax.ShapeDtypeStruct((_N, _HID), jnp.float32),
  )(h, agg, w1, b1.reshape(1, -1), w2, b2.reshape(1, -1),
    gamma.reshape(1, -1), beta.reshape(1, -1))


def _last_body(h2_ref, a_ref, w1_ref, b1_ref, w2_ref, b2_ref, g_ref, be_ref,
               h1_ref, b2d_ref, wj_ref, bj_ref, wc1_ref, bc1_ref,
               wc2_ref, bc2_ref, o_ref):
  h2 = h2_ref[...]
  h3 = _mlp(h2, a_ref[0, :_N], a_ref[1, :_N], w1_ref[...],
            b1_ref[...], w2_ref[...], b2_ref[...], g_ref[...], be_ref[...])
  hc = jnp.concatenate([h1_ref[...], h2, h3], axis=1)
  hj = jnp.dot(hc, wj_ref[...], preferred_element_type=jnp.float32) + bj_ref[...]
  gids = lax.broadcasted_iota(jnp.int32, (_NG, _N), 0)
  mask = (gids == b2d_ref[...]).astype(jnp.float32)
  pooled = jnp.dot(mask, hj, preferred_element_type=jnp.float32,
                   precision=lax.Precision.HIGHEST)
  cmid = jnp.maximum(
      jnp.dot(pooled, wc1_ref[...], preferred_element_type=jnp.float32)
      + bc1_ref[...], 0.0)
  o_ref[...] = (jnp.dot(cmid, wc2_ref[...], preferred_element_type=jnp.float32)
                + bc2_ref[...])


def _last_call(h2, agg, w1, b1, w2, b2, gamma, beta, h1, batch2d, wj, bj,
               wc1, bc1, wc2, bc2):
  return pl.pallas_call(
      _last_body,
      out_shape=jax.ShapeDtypeStruct((_NG, _NCLS), jnp.float32),
  )(h2, agg, w1, b1.reshape(1, -1), w2, b2.reshape(1, -1),
    gamma.reshape(1, -1), beta.reshape(1, -1), h1, batch2d, wj,
    bj.reshape(1, -1), wc1, bc1.reshape(1, -1), wc2, bc2.reshape(1, -1))


def kernel(x, edge_index, batch, params):
  edge3d = edge_index.reshape(2, _E // _CHUNK, _CHUNK)
  batch2d = batch.reshape(1, _N)
  zeros128 = jnp.zeros((_NPAD, _F_IN), jnp.float32)
  zeros64 = jnp.zeros((_NPAD, _HID), jnp.float32)

  agg = _agg128(x, edge3d, zeros128)
  h1 = _mlp_call(x, agg, params['W1_0'], params['b1_0'], params['W2_0'],
                 params['b2_0'], params['gamma_0'], params['beta_0'])
  agg = _agg64(h1, edge3d, zeros64)
  h2 = _mlp_call(h1, agg, params['W1_1'], params['b1_1'], params['W2_1'],
                 params['b2_1'], params['gamma_1'], params['beta_1'])
  agg = _agg64(h2, edge3d, zeros64)
  return _last_call(h2, agg, params['W1_2'], params['b1_2'], params['W2_2'],
                    params['b2_2'], params['gamma_2'], params['beta_2'],
                    h1, batch2d, params['Wj'], params['bj'],
                    params['Wc1'], params['bc1'], params['Wc2'],
                    params['bc2'])
